# baseline (speedup 1.0000x reference)
import functools

import jax
import jax.numpy as jnp
from jax import lax
from jax.experimental import pallas as pl
from jax.experimental.pallas import tpu as pltpu
from jax.experimental.pallas import tpu_sc as plsc

N = 65536
D = 128
NT = 32
NCT = 64
L = 16
NC = 2
NS = 16
NW = NC * NS
NPW = N // NW
C = 32
NCHUNKS = NPW // C

_mesh = plsc.VectorSubcoreMesh(core_axis_name="c", subcore_axis_name="s")


@functools.partial(
    pl.kernel,
    mesh=_mesh,
    out_type=[
        jax.ShapeDtypeStruct((N, D), jnp.float32),
        jax.ShapeDtypeStruct((3, N, D), jnp.float32),
    ],
    compiler_params=pltpu.CompilerParams(needs_layout_passes=False),
    scratch_types=[
        pltpu.VMEM((NPW,), jnp.int32),
        pltpu.VMEM((NPW,), jnp.int32),
        pltpu.VMEM((9 * NPW,), jnp.float32),
        pltpu.VMEM((2, C, D), jnp.float32),
        pltpu.VMEM((2, C, D), jnp.float32),
        pltpu.VMEM((2, C, 3 * D), jnp.float32),
        pltpu.VMEM((2, C, D), jnp.float32),
        pltpu.VMEM((2, 3, C, D), jnp.float32),
        pltpu.SemaphoreType.DMA,
        pltpu.SemaphoreType.DMA,
        pltpu.SemaphoreType.DMA,
        pltpu.SemaphoreType.DMA,
        pltpu.SemaphoreType.DMA,
    ],
)
def _sc_embed(nt_hbm, cid_hbm, rot_hbm, ts_hbm, cs_hbm, tvp_hbm,
              s_hbm, v_hbm,
              nt_v, cid_v, rot_v,
              a_buf, b_buf, pv_buf, s_buf, v_buf,
              sem_in, gsem0, gsem1, osem0, osem1):
    wid = lax.axis_index("s") * NC + lax.axis_index("c")
    base = wid * NPW

    copies = [
        pltpu.async_copy(nt_hbm.at[pl.ds(base, NPW)], nt_v, sem_in),
        pltpu.async_copy(cid_hbm.at[pl.ds(base, NPW)], cid_v, sem_in),
    ] + [
        pltpu.async_copy(rot_hbm.at[pl.ds(k * N + base, NPW)],
                         rot_v.at[pl.ds(k * NPW, NPW)], sem_in)
        for k in range(9)
    ]
    for cp in copies:
        cp.wait()

    def issue_gathers(g, b, gsem):
        nt_idx = nt_v.at[pl.ds(g * C, C)]
        cid_idx = cid_v.at[pl.ds(g * C, C)]
        pltpu.async_copy(ts_hbm.at[nt_idx], a_buf.at[b], gsem)
        pltpu.async_copy(cs_hbm.at[cid_idx], b_buf.at[b], gsem)
        pltpu.async_copy(tvp_hbm.at[nt_idx], pv_buf.at[b], gsem)

    def wait_gathers(g, b, gsem):
        nt_idx = nt_v.at[pl.ds(g * C, C)]
        cid_idx = cid_v.at[pl.ds(g * C, C)]
        pltpu.make_async_copy(ts_hbm.at[nt_idx], a_buf.at[b], gsem).wait()
        pltpu.make_async_copy(cs_hbm.at[cid_idx], b_buf.at[b], gsem).wait()
        pltpu.make_async_copy(tvp_hbm.at[nt_idx], pv_buf.at[b], gsem).wait()

    def issue_outs(g, b, osem):
        nbase = base + g * C
        pltpu.async_copy(s_buf.at[b], s_hbm.at[pl.ds(nbase, C)], osem)
        for i in range(3):
            pltpu.async_copy(v_buf.at[b, i],
                             v_hbm.at[i, pl.ds(nbase, C)], osem)

    def wait_outs(g, b, osem):
        nbase = base + g * C
        pltpu.make_async_copy(
            s_buf.at[b], s_hbm.at[pl.ds(nbase, C)], osem).wait()
        for i in range(3):
            pltpu.make_async_copy(v_buf.at[b, i],
                                  v_hbm.at[i, pl.ds(nbase, C)], osem).wait()

    def compute(g, b):
        @plsc.parallel_loop(0, C // L, 1)
        def group_body(nb):
            gb = nb * L
            lb = g * C + gb
            for m in range(L):
                row = gb + m
                for cb in range(D // L):
                    s_buf[b, row, pl.ds(cb * L, L)] = (
                        a_buf[b, row, pl.ds(cb * L, L)]
                        + b_buf[b, row, pl.ds(cb * L, L)])
                r = [plsc.load_gather(
                        rot_v, [jnp.full((L,), k * NPW + lb + m, jnp.int32)])
                     for k in range(9)]
                for db in range(D // L):
                    p = [pv_buf[b, row, pl.ds(j * D + db * L, L)]
                         for j in range(3)]
                    for i in range(3):
                        o = r[3 * i] * p[0] + r[3 * i + 1] * p[1] \
                            + r[3 * i + 2] * p[2]
                        v_buf[b, i, row, pl.ds(db * L, L)] = o

    issue_gathers(0, 0, gsem0)
    issue_gathers(1, 1, gsem1)

    def step(g, b, gsem, osem):
        wait_gathers(g, b, gsem)

        @pl.when(g >= 2)
        def _():
            wait_outs(g - 2, b, osem)

        compute(g, b)
        issue_outs(g, b, osem)

        @pl.when(g + 2 < NCHUNKS)
        def _():
            issue_gathers(g + 2, b, gsem)

    def half_body(h, carry):
        step(2 * h, 0, gsem0, osem0)
        step(2 * h + 1, 1, gsem1, osem1)
        return carry

    lax.fori_loop(0, NCHUNKS // 2, half_body, 0)

    wait_outs(NCHUNKS - 2, 0, osem0)
    wait_outs(NCHUNKS - 1, 1, osem1)


def kernel(node_type, rotmat, chain_id, type2scalar, type2vec, chain2scalar):
    nt = node_type.astype(jnp.int32)
    cid = chain_id.astype(jnp.int32)
    rot = rotmat.transpose(1, 2, 0).reshape(9 * N)
    ts = type2scalar
    cs = chain2scalar
    tvp = type2vec.reshape(NT, D, 3).transpose(0, 2, 1).reshape(NT, 3 * D)
    s_out, v_out = _sc_embed(nt, cid, rot, ts, cs, tvp)
    return s_out, v_out.transpose(1, 2, 0)


# speedup vs baseline: 2.5553x; 2.5553x over previous
import functools

import jax
import jax.numpy as jnp
from jax import lax
from jax.experimental import pallas as pl
from jax.experimental.pallas import tpu as pltpu
from jax.experimental.pallas import tpu_sc as plsc

N = 65536
D = 128
NT = 32
NCT = 64

L = 16
NC = 2
NS = 16
NW = NC * NS
NPW = N // NW
C = 64
NCHUNKS = NPW // C

B = 512
NBLK = N // B

_mesh = plsc.VectorSubcoreMesh(core_axis_name="c", subcore_axis_name="s")


@functools.partial(
    pl.kernel,
    mesh=_mesh,
    out_type=jax.ShapeDtypeStruct((N, D), jnp.float32),
    compiler_params=pltpu.CompilerParams(needs_layout_passes=False),
    scratch_types=[
        pltpu.VMEM((NPW,), jnp.int32),
        pltpu.VMEM((NPW,), jnp.int32),
        pltpu.VMEM((2, C, D), jnp.float32),
        pltpu.VMEM((2, C, D), jnp.float32),
        pltpu.VMEM((2, C, D), jnp.float32),
        pltpu.SemaphoreType.DMA,
        pltpu.SemaphoreType.DMA,
        pltpu.SemaphoreType.DMA,
        pltpu.SemaphoreType.DMA,
        pltpu.SemaphoreType.DMA,
    ],
)
def _sc_scalar(nt_hbm, cid_hbm, ts_hbm, cs_hbm, s_hbm,
               nt_v, cid_v, a_buf, b_buf, s_buf,
               sem_in, gsem0, gsem1, osem0, osem1):
    wid = lax.axis_index("s") * NC + lax.axis_index("c")
    base = wid * NPW

    copies = [
        pltpu.async_copy(nt_hbm.at[pl.ds(base, NPW)], nt_v, sem_in),
        pltpu.async_copy(cid_hbm.at[pl.ds(base, NPW)], cid_v, sem_in),
    ]
    for cp in copies:
        cp.wait()

    def issue_gathers(g, b, gsem):
        nt_idx = nt_v.at[pl.ds(g * C, C)]
        cid_idx = cid_v.at[pl.ds(g * C, C)]
        pltpu.async_copy(ts_hbm.at[nt_idx], a_buf.at[b], gsem)
        pltpu.async_copy(cs_hbm.at[cid_idx], b_buf.at[b], gsem)

    def wait_gathers(g, b, gsem):
        nt_idx = nt_v.at[pl.ds(g * C, C)]
        cid_idx = cid_v.at[pl.ds(g * C, C)]
        pltpu.make_async_copy(ts_hbm.at[nt_idx], a_buf.at[b], gsem).wait()
        pltpu.make_async_copy(cs_hbm.at[cid_idx], b_buf.at[b], gsem).wait()

    def issue_out(g, b, osem):
        pltpu.async_copy(s_buf.at[b], s_hbm.at[pl.ds(base + g * C, C)], osem)

    def wait_out(g, b, osem):
        pltpu.make_async_copy(
            s_buf.at[b], s_hbm.at[pl.ds(base + g * C, C)], osem).wait()

    def compute(g, b):
        @plsc.parallel_loop(0, C // L, 1)
        def group_body(nb):
            gb = nb * L
            for m in range(L):
                row = gb + m
                for cb in range(D // L):
                    s_buf[b, row, pl.ds(cb * L, L)] = (
                        a_buf[b, row, pl.ds(cb * L, L)]
                        + b_buf[b, row, pl.ds(cb * L, L)])

    issue_gathers(0, 0, gsem0)
    issue_gathers(1, 1, gsem1)

    def step(g, b, gsem, osem):
        wait_gathers(g, b, gsem)

        @pl.when(g >= 2)
        def _():
            wait_out(g - 2, b, osem)

        compute(g, b)
        issue_out(g, b, osem)

        @pl.when(g + 2 < NCHUNKS)
        def _():
            issue_gathers(g + 2, b, gsem)

    def half_body(h, carry):
        step(2 * h, 0, gsem0, osem0)
        step(2 * h + 1, 1, gsem1, osem1)
        return carry

    lax.fori_loop(0, NCHUNKS // 2, half_body, 0)

    wait_out(NCHUNKS - 2, 0, osem0)
    wait_out(NCHUNKS - 1, 1, osem1)


def _tc_vector_body(nt_ref, rot_ref, tvp_ref, vout_ref):
    nt = nt_ref[0]
    iota = lax.broadcasted_iota(jnp.int32, (NT, B), 0)
    oh = jnp.where(iota == nt, 1.0, 0.0).astype(jnp.float32)
    vp = lax.dot_general(oh, tvp_ref[...],
                         dimension_numbers=(((0,), (0,)), ((), ())),
                         preferred_element_type=jnp.float32)
    eye3 = jnp.eye(3, dtype=jnp.float32)
    for i in range(3):
        rt = lax.dot_general(rot_ref[i], eye3,
                             dimension_numbers=(((0,), (0,)), ((), ())),
                             preferred_element_type=jnp.float32)
        o = (rt[:, 0:1] * vp[:, 0 * D:1 * D]
             + rt[:, 1:2] * vp[:, 1 * D:2 * D]
             + rt[:, 2:3] * vp[:, 2 * D:3 * D])
        vout_ref[i] = o


_tc_vector = pl.pallas_call(
    _tc_vector_body,
    grid=(NBLK,),
    in_specs=[
        pl.BlockSpec((1, 1, B), lambda nb: (nb, 0, 0)),
        pl.BlockSpec((3, 3, B), lambda nb: (0, 0, nb)),
        pl.BlockSpec((NT, 3 * D), lambda nb: (0, 0)),
    ],
    out_specs=pl.BlockSpec((3, B, D), lambda nb: (0, nb, 0)),
    out_shape=jax.ShapeDtypeStruct((3, N, D), jnp.float32),
)


def kernel(node_type, rotmat, chain_id, type2scalar, type2vec, chain2scalar):
    nt = node_type.astype(jnp.int32)
    cid = chain_id.astype(jnp.int32)
    rotp = rotmat.transpose(1, 2, 0)
    tvp = type2vec.reshape(NT, D, 3).transpose(0, 2, 1).reshape(NT, 3 * D)
    s_out = _sc_scalar(nt, cid, type2scalar, chain2scalar)
    v_out = _tc_vector(nt.reshape(NBLK, 1, B), rotp, tvp)
    return s_out, v_out.transpose(1, 2, 0)


# speedup vs baseline: 4.8647x; 1.9038x over previous
import functools

import jax
import jax.numpy as jnp
from jax import lax
from jax.experimental import pallas as pl
from jax.experimental.pallas import tpu as pltpu
from jax.experimental.pallas import tpu_sc as plsc

N = 65536
D = 128
NT = 32
NCT = 64

L = 16
NC = 2
NS = 16
NW = NC * NS
NPW = N // NW
C = 128
NCHUNKS = NPW // C
NSLOT = 4

B = 1024
NBLK = N // B

_mesh = plsc.VectorSubcoreMesh(core_axis_name="c", subcore_axis_name="s")


@functools.partial(
    pl.kernel,
    mesh=_mesh,
    out_type=jax.ShapeDtypeStruct((N, D), jnp.float32),
    compiler_params=pltpu.CompilerParams(needs_layout_passes=False),
    scratch_types=[
        pltpu.VMEM((NPW,), jnp.int32),
        pltpu.VMEM((NPW,), jnp.int32),
        pltpu.VMEM((NPW,), jnp.int32),
        pltpu.VMEM((NSLOT, C, D), jnp.float32),
        pltpu.SemaphoreType.DMA,
    ] + [pltpu.SemaphoreType.DMA] * NSLOT
      + [pltpu.SemaphoreType.DMA] * NSLOT,
)
def _sc_scalar(nt_hbm, cid_hbm, comb_hbm, s_hbm,
               nt_v, cid_v, idx_v, s_buf, sem_in, *sems):
    gsems, osems = sems[:NSLOT], sems[NSLOT:]
    wid = lax.axis_index("s") * NC + lax.axis_index("c")
    base = wid * NPW

    copies = [
        pltpu.async_copy(nt_hbm.at[pl.ds(base, NPW)], nt_v, sem_in),
        pltpu.async_copy(cid_hbm.at[pl.ds(base, NPW)], cid_v, sem_in),
    ]
    for cp in copies:
        cp.wait()

    @plsc.parallel_loop(0, NPW // L, 1)
    def idx_body(q):
        qb = q * L
        idx_v[pl.ds(qb, L)] = nt_v[pl.ds(qb, L)] * NCT + cid_v[pl.ds(qb, L)]

    def gather(g, b):
        return pltpu.make_async_copy(
            comb_hbm.at[idx_v.at[pl.ds(g * C, C)]], s_buf.at[b], gsems[b])

    def out(g, b):
        return pltpu.make_async_copy(
            s_buf.at[b], s_hbm.at[pl.ds(base + g * C, C)], osems[b])

    def substep(g, b):
        @pl.when(jnp.logical_and(g >= NSLOT, g - NSLOT < NCHUNKS))
        def _():
            out(g - NSLOT, b).wait()

        @pl.when(g < NCHUNKS)
        def _():
            gather(g, b).start()

        g2 = g - 2
        b2 = (b + NSLOT - 2) % NSLOT

        @pl.when(jnp.logical_and(g2 >= 0, g2 < NCHUNKS))
        def _():
            gather(g2, b2).wait()
            out(g2, b2).start()

    def ring_body(h, carry):
        for sub in range(NSLOT):
            substep(h * NSLOT + sub, sub)
        return carry

    lax.fori_loop(0, (NCHUNKS + NSLOT) // NSLOT + 1, ring_body, 0)


def _tc_vector_body(nt_ref, rot_ref, tvp_ref, vout_ref):
    nt = nt_ref[0]
    iota = lax.broadcasted_iota(jnp.int32, (NT, B), 0)
    oh = jnp.where(iota == nt, 1.0, 0.0).astype(jnp.float32)
    vp = lax.dot_general(oh, tvp_ref[...],
                         dimension_numbers=(((0,), (0,)), ((), ())),
                         preferred_element_type=jnp.float32)
    e = (lax.broadcasted_iota(jnp.int32, (9, 9 * D), 0)
         == lax.broadcasted_iota(jnp.int32, (9, 9 * D), 1) // D
         ).astype(jnp.float32)
    rb = lax.dot_general(rot_ref[...], e,
                         dimension_numbers=(((0,), (0,)), ((), ())),
                         preferred_element_type=jnp.float32)
    for i in range(3):
        o = (rb[:, (3 * i + 0) * D:(3 * i + 1) * D] * vp[:, 0 * D:1 * D]
             + rb[:, (3 * i + 1) * D:(3 * i + 2) * D] * vp[:, 1 * D:2 * D]
             + rb[:, (3 * i + 2) * D:(3 * i + 3) * D] * vp[:, 2 * D:3 * D])
        vout_ref[i] = o


_tc_vector = pl.pallas_call(
    _tc_vector_body,
    grid=(NBLK,),
    in_specs=[
        pl.BlockSpec((1, 1, B), lambda nb: (nb, 0, 0)),
        pl.BlockSpec((9, B), lambda nb: (0, nb)),
        pl.BlockSpec((NT, 3 * D), lambda nb: (0, 0)),
    ],
    out_specs=pl.BlockSpec((3, B, D), lambda nb: (0, nb, 0)),
    out_shape=jax.ShapeDtypeStruct((3, N, D), jnp.float32),
)


def kernel(node_type, rotmat, chain_id, type2scalar, type2vec, chain2scalar):
    nt = node_type.astype(jnp.int32)
    cid = chain_id.astype(jnp.int32)
    rotp = rotmat.transpose(1, 2, 0).reshape(9, N)
    comb = (type2scalar[:, None, :] + chain2scalar[None, :, :]).reshape(
        NT * NCT, D)
    tvp = type2vec.reshape(NT, D, 3).transpose(0, 2, 1).reshape(NT, 3 * D)
    s_out = _sc_scalar(nt, cid, comb)
    v_out = _tc_vector(nt.reshape(NBLK, 1, B), rotp, tvp)
    return s_out, v_out.transpose(1, 2, 0)


# speedup vs baseline: 5.3597x; 1.1017x over previous
import functools

import jax
import jax.numpy as jnp
from jax import lax
from jax.experimental import pallas as pl
from jax.experimental.pallas import tpu as pltpu
from jax.experimental.pallas import tpu_sc as plsc

N = 65536
D = 128
NT = 32
NCT = 64

L = 16
NC = 2
NS = 16
NW = NC * NS
NPW = N // NW
C = 128
NCHUNKS = NPW // C
NSLOT = 4

B = 2048
NBLK = N // B

_mesh = plsc.VectorSubcoreMesh(core_axis_name="c", subcore_axis_name="s")


@functools.partial(
    pl.kernel,
    mesh=_mesh,
    out_type=jax.ShapeDtypeStruct((N, D), jnp.float32),
    compiler_params=pltpu.CompilerParams(needs_layout_passes=False),
    scratch_types=[
        pltpu.VMEM((NPW,), jnp.int32),
        pltpu.VMEM((NPW,), jnp.int32),
        pltpu.VMEM((NPW,), jnp.int32),
        pltpu.VMEM((NSLOT, C, D), jnp.float32),
        pltpu.SemaphoreType.DMA,
    ] + [pltpu.SemaphoreType.DMA] * NSLOT
      + [pltpu.SemaphoreType.DMA] * NSLOT,
)
def _sc_scalar(nt_hbm, cid_hbm, comb_hbm, s_hbm,
               nt_v, cid_v, idx_v, s_buf, sem_in, *sems):
    gsems, osems = sems[:NSLOT], sems[NSLOT:]
    wid = lax.axis_index("s") * NC + lax.axis_index("c")
    base = wid * NPW

    copies = [
        pltpu.async_copy(nt_hbm.at[pl.ds(base, NPW)], nt_v, sem_in),
        pltpu.async_copy(cid_hbm.at[pl.ds(base, NPW)], cid_v, sem_in),
    ]
    for cp in copies:
        cp.wait()

    @plsc.parallel_loop(0, NPW // L, 1)
    def idx_body(q):
        qb = q * L
        idx_v[pl.ds(qb, L)] = nt_v[pl.ds(qb, L)] * NCT + cid_v[pl.ds(qb, L)]

    def gather(g, b):
        return pltpu.make_async_copy(
            comb_hbm.at[idx_v.at[pl.ds(g * C, C)]], s_buf.at[b], gsems[b])

    def out(g, b):
        return pltpu.make_async_copy(
            s_buf.at[b], s_hbm.at[pl.ds(base + g * C, C)], osems[b])

    def substep(g, b):
        @pl.when(jnp.logical_and(g >= NSLOT, g - NSLOT < NCHUNKS))
        def _():
            out(g - NSLOT, b).wait()

        @pl.when(g < NCHUNKS)
        def _():
            gather(g, b).start()

        g2 = g - 2
        b2 = (b + NSLOT - 2) % NSLOT

        @pl.when(jnp.logical_and(g2 >= 0, g2 < NCHUNKS))
        def _():
            gather(g2, b2).wait()
            out(g2, b2).start()

    def ring_body(h, carry):
        for sub in range(NSLOT):
            substep(h * NSLOT + sub, sub)
        return carry

    lax.fori_loop(0, (NCHUNKS + NSLOT) // NSLOT + 1, ring_body, 0)


def _tc_vector_body(nt_ref, rot_ref, tvp_ref, vout_ref):
    nt = nt_ref[0]
    iota = lax.broadcasted_iota(jnp.int32, (NT, B), 0)
    oh = jnp.where(iota == nt, 1.0, 0.0).astype(jnp.float32)
    vp = lax.dot_general(oh, tvp_ref[...],
                         dimension_numbers=(((0,), (0,)), ((), ())),
                         preferred_element_type=jnp.float32)
    e = (lax.broadcasted_iota(jnp.int32, (9, 9 * D), 0)
         == lax.broadcasted_iota(jnp.int32, (9, 9 * D), 1) // D
         ).astype(jnp.float32)
    rb = lax.dot_general(rot_ref[...], e,
                         dimension_numbers=(((0,), (0,)), ((), ())),
                         preferred_element_type=jnp.float32)
    for i in range(3):
        o = (rb[:, (3 * i + 0) * D:(3 * i + 1) * D] * vp[:, 0 * D:1 * D]
             + rb[:, (3 * i + 1) * D:(3 * i + 2) * D] * vp[:, 1 * D:2 * D]
             + rb[:, (3 * i + 2) * D:(3 * i + 3) * D] * vp[:, 2 * D:3 * D])
        vout_ref[i] = o


_tc_vector = pl.pallas_call(
    _tc_vector_body,
    grid=(NBLK,),
    in_specs=[
        pl.BlockSpec((1, 1, B), lambda nb: (nb, 0, 0)),
        pl.BlockSpec((9, B), lambda nb: (0, nb)),
        pl.BlockSpec((NT, 3 * D), lambda nb: (0, 0)),
    ],
    out_specs=pl.BlockSpec((3, B, D), lambda nb: (0, nb, 0)),
    out_shape=jax.ShapeDtypeStruct((3, N, D), jnp.float32),
)


def kernel(node_type, rotmat, chain_id, type2scalar, type2vec, chain2scalar):
    nt = node_type.astype(jnp.int32)
    cid = chain_id.astype(jnp.int32)
    rotp = rotmat.transpose(1, 2, 0).reshape(9, N)
    comb = (type2scalar[:, None, :] + chain2scalar[None, :, :]).reshape(
        NT * NCT, D)
    tvp = type2vec.reshape(NT, D, 3).transpose(0, 2, 1).reshape(NT, 3 * D)
    s_out = _sc_scalar(nt, cid, comb)
    v_out = _tc_vector(nt.reshape(NBLK, 1, B), rotp, tvp)
    return s_out, v_out.transpose(1, 2, 0)
